# baseline (speedup 1.0000x reference)
import functools

import jax
import jax.numpy as jnp
from jax import lax
from jax.experimental import pallas as pl
from jax.experimental.pallas import tpu as pltpu
from jax.experimental.pallas import tpu_sc as plsc

N, D = 16384, 128
NC, NS = 2, 16
NW = NC * NS
B_W = N // NW
CHUNK = 64
NCH = B_W // CHUNK

with jax.default_device(jax.devices("cpu")[0]):
    _PERM = jax.random.permutation(jax.random.key(42), N)
    _INV = jnp.argsort(_PERM).astype(jnp.int32).reshape(NW, NCH, CHUNK)


def _shuffle_body(x_hbm, idx_hbm, out_hbm, idx_v, rows_v,
                  s0, s1, s2, s3, s4, s5, s6, s7, s_out):
    wid = lax.axis_index("s") * NC + lax.axis_index("c")
    base = wid * B_W
    sems = (s0, s1, s2, s3, s4, s5, s6, s7)
    reads = [
        pltpu.async_copy(
            x_hbm.at[pl.ds(base + c * CHUNK, CHUNK)],
            rows_v.at[pl.ds(c * CHUNK, CHUNK)],
            sems[c],
        )
        for c in range(NCH)
    ]
    pltpu.sync_copy(idx_hbm.at[wid], idx_v)
    scatters = []
    for c in range(NCH):
        reads[c].wait()
        scatters.append(
            pltpu.async_copy(
                rows_v.at[pl.ds(c * CHUNK, CHUNK)],
                out_hbm.at[idx_v.at[c]],
                s_out,
            )
        )
    for cp in scatters:
        cp.wait()


_shuffle = functools.partial(
    pl.kernel,
    mesh=plsc.VectorSubcoreMesh(core_axis_name="c", subcore_axis_name="s"),
    out_type=jax.ShapeDtypeStruct((N, D), jnp.float32),
    scratch_types=[
        pltpu.VMEM((NCH, CHUNK), jnp.int32),
        pltpu.VMEM((B_W, D), jnp.float32),
        pltpu.SemaphoreType.DMA,
        pltpu.SemaphoreType.DMA,
        pltpu.SemaphoreType.DMA,
        pltpu.SemaphoreType.DMA,
        pltpu.SemaphoreType.DMA,
        pltpu.SemaphoreType.DMA,
        pltpu.SemaphoreType.DMA,
        pltpu.SemaphoreType.DMA,
        pltpu.SemaphoreType.DMA,
    ],
)(_shuffle_body)


def kernel(x):
    return _shuffle(x, _INV)


# speedup vs baseline: 1.0226x; 1.0226x over previous
import functools

import jax
import jax.numpy as jnp
from jax import lax
from jax.experimental import pallas as pl
from jax.experimental.pallas import tpu as pltpu
from jax.experimental.pallas import tpu_sc as plsc

N, D = 16384, 128
NC, NS = 2, 16
NW = NC * NS
B_W = N // NW
CHUNK = 128
NCH = B_W // CHUNK

with jax.default_device(jax.devices("cpu")[0]):
    _PERM = jax.random.permutation(jax.random.key(42), N)
    _INV = jnp.argsort(_PERM).astype(jnp.int32).reshape(NW, NCH, CHUNK)


def _shuffle_body(x_hbm, idx_hbm, out_hbm, idx_v, rows_v, s0, s1, s2, s3, s_out):
    wid = lax.axis_index("s") * NC + lax.axis_index("c")
    base = wid * B_W
    sems = (s0, s1, s2, s3)
    reads = [
        pltpu.async_copy(
            x_hbm.at[pl.ds(base + c * CHUNK, CHUNK)],
            rows_v.at[pl.ds(c * CHUNK, CHUNK)],
            sems[c],
        )
        for c in range(NCH)
    ]
    pltpu.sync_copy(idx_hbm.at[wid], idx_v)
    scatters = []
    for c in range(NCH):
        reads[c].wait()
        scatters.append(
            pltpu.async_copy(
                rows_v.at[pl.ds(c * CHUNK, CHUNK)],
                out_hbm.at[idx_v.at[c]],
                s_out,
            )
        )
    for cp in scatters:
        cp.wait()


_shuffle = functools.partial(
    pl.kernel,
    mesh=plsc.VectorSubcoreMesh(core_axis_name="c", subcore_axis_name="s"),
    out_type=jax.ShapeDtypeStruct((N, D), jnp.float32),
    scratch_types=[
        pltpu.VMEM((NCH, CHUNK), jnp.int32),
        pltpu.VMEM((B_W, D), jnp.float32),
        pltpu.SemaphoreType.DMA,
        pltpu.SemaphoreType.DMA,
        pltpu.SemaphoreType.DMA,
        pltpu.SemaphoreType.DMA,
        pltpu.SemaphoreType.DMA,
    ],
)(_shuffle_body)


def kernel(x):
    return _shuffle(x, _INV)
